# Initial kernel scaffold; baseline (speedup 1.0000x reference)
import functools

import jax
import jax.numpy as jnp
from jax.experimental import pallas as pl

N = 10000
N_PAD = 10240
E = 320000
D_FEAT = 128
CH = 32
K = 6000


def _gelu(x):
    return jax.nn.gelu(x, approximate=False)


def _dense0_body(x_ref, w_ref, b_ref, o_ref):
    y = jax.lax.dot_general(x_ref[...], w_ref[...], (((1,), (1,)), ((), ())),
                            preferred_element_type=jnp.float32)
    o_ref[...] = _gelu(y + b_ref[...])


def _dense0(x, w, b, block_rows=2048):
    n = x.shape[0]
    dout = w.shape[0]
    return pl.pallas_call(
        _dense0_body,
        grid=(n // block_rows,),
        in_specs=[
            pl.BlockSpec((block_rows, x.shape[1]), lambda i: (i, 0)),
            pl.BlockSpec((w.shape[0], w.shape[1]), lambda i: (0, 0)),
            pl.BlockSpec((1, dout), lambda i: (0, 0)),
        ],
        out_specs=pl.BlockSpec((block_rows, dout), lambda i: (i, 0)),
        out_shape=jax.ShapeDtypeStruct((n, dout), jnp.float32),
    )(x, w, b.reshape(1, -1))


def _dense_body(agg_ref, h_ref, wrel_ref, b_ref, wroot_ref, o_ref):
    a = agg_ref[0] + agg_ref[1]
    y = jax.lax.dot_general(a, wrel_ref[...], (((1,), (1,)), ((), ())),
                            preferred_element_type=jnp.float32)
    y = y + jax.lax.dot_general(h_ref[...], wroot_ref[...], (((1,), (1,)), ((), ())),
                                preferred_element_type=jnp.float32)
    o_ref[...] = _gelu(y + b_ref[...])


def _dense(agg2, h, wrel, b, wroot, block_rows=2048):
    n = h.shape[0]
    din = h.shape[1]
    dout = wrel.shape[0]
    return pl.pallas_call(
        _dense_body,
        grid=(n // block_rows,),
        in_specs=[
            pl.BlockSpec((2, block_rows, din), lambda i: (0, i, 0)),
            pl.BlockSpec((block_rows, din), lambda i: (i, 0)),
            pl.BlockSpec((dout, din), lambda i: (0, 0)),
            pl.BlockSpec((1, dout), lambda i: (0, 0)),
            pl.BlockSpec((dout, din), lambda i: (0, 0)),
        ],
        out_specs=pl.BlockSpec((block_rows, dout), lambda i: (i, 0)),
        out_shape=jax.ShapeDtypeStruct((n, dout), jnp.float32),
    )(agg2, h, wrel, b.reshape(1, -1), wroot)


def _agg_jnp(h, src, dst):
    out = jnp.zeros((2, h.shape[0], h.shape[1]), h.dtype)
    return out.at[0, dst].add(h[src])


def kernel(x, edge_index, Wd_rel, bd_rel, Wd_root, W0_rel, b0_rel, W0_root,
           W1_rel, b1_rel, W1_root, W2_rel, b2_rel, W2_root, p,
           W3_rel, b3_rel, W3_root, Wout, bout):
    src, dst = edge_index[0], edge_index[1]
    x_pad = jnp.zeros((N_PAD, D_FEAT), jnp.float32).at[:N].set(x)

    h0 = _dense0(x_pad, Wd_root, bd_rel)
    agg0 = _agg_jnp(h0, src, dst)
    h1 = _dense(agg0, h0, W0_rel, b0_rel, W0_root)
    agg1 = _agg_jnp(h1, src, dst)
    h2 = _dense(agg1, h1, W1_rel, b1_rel, W1_root)
    agg2 = _agg_jnp(h2, src, dst)
    h3 = _dense(agg2, h2, W2_rel, b2_rel, W2_root)

    score = jnp.tanh((h3[:N] @ p) / jnp.linalg.norm(p))
    idx = jnp.arange(N)
    gt = (score[:, None] > score[None, :]) | (
        (score[:, None] == score[None, :]) & (idx[:, None] < idx[None, :]))
    rank = jnp.sum(gt.astype(jnp.int32), axis=0)
    kept = rank < K
    parity = rank % 2

    hp = jnp.where(kept[:, None], h3[:N] * score[:, None], 0.0)
    hp = jnp.zeros((N_PAD, D_FEAT), jnp.float32).at[:N].set(hp)
    agg3 = _agg_jnp(hp, src, dst)
    h4 = _dense(agg3, hp, W3_rel, b3_rel, W3_root)

    w_even = (kept & (parity == 0)).astype(jnp.float32)
    w_odd = (kept & (parity == 1)).astype(jnp.float32)
    wsel = jnp.zeros((2, N_PAD), jnp.float32)
    wsel = wsel.at[0, :N].set(w_even).at[1, :N].set(w_odd)
    pooled = (wsel @ h4) / (K // 2)
    return Wout @ pooled.reshape(-1) + bout


# speedup vs baseline: 2.8780x; 2.8780x over previous
import functools

import jax
import jax.numpy as jnp
from jax.experimental import pallas as pl

N = 10000
N_PAD = 10240
E = 320000
D_FEAT = 128
CH = 32
K = 6000


def _gelu(x):
    return 0.5 * x * (1.0 + jax.lax.erf(x * 0.7071067811865476))


def _dense0_body(x_ref, w_ref, b_ref, o_ref):
    y = jax.lax.dot_general(x_ref[...], w_ref[...], (((1,), (1,)), ((), ())),
                            preferred_element_type=jnp.float32)
    o_ref[...] = _gelu(y + b_ref[...])


def _dense0(x, w, b, block_rows=2048):
    n = x.shape[0]
    dout = w.shape[0]
    return pl.pallas_call(
        _dense0_body,
        grid=(n // block_rows,),
        in_specs=[
            pl.BlockSpec((block_rows, x.shape[1]), lambda i: (i, 0)),
            pl.BlockSpec((w.shape[0], w.shape[1]), lambda i: (0, 0)),
            pl.BlockSpec((1, dout), lambda i: (0, 0)),
        ],
        out_specs=pl.BlockSpec((block_rows, dout), lambda i: (i, 0)),
        out_shape=jax.ShapeDtypeStruct((n, dout), jnp.float32),
    )(x, w, b.reshape(1, -1))


def _dense_body(agg_ref, h_ref, wrel_ref, b_ref, wroot_ref, o_ref):
    a = agg_ref[0] + agg_ref[1]
    y = jax.lax.dot_general(a, wrel_ref[...], (((1,), (1,)), ((), ())),
                            preferred_element_type=jnp.float32)
    y = y + jax.lax.dot_general(h_ref[...], wroot_ref[...], (((1,), (1,)), ((), ())),
                                preferred_element_type=jnp.float32)
    o_ref[...] = _gelu(y + b_ref[...])


def _dense(agg2, h, wrel, b, wroot, block_rows=2048):
    n = h.shape[0]
    din = h.shape[1]
    dout = wrel.shape[0]
    return pl.pallas_call(
        _dense_body,
        grid=(n // block_rows,),
        in_specs=[
            pl.BlockSpec((2, block_rows, din), lambda i: (0, i, 0)),
            pl.BlockSpec((block_rows, din), lambda i: (i, 0)),
            pl.BlockSpec((dout, din), lambda i: (0, 0)),
            pl.BlockSpec((1, dout), lambda i: (0, 0)),
            pl.BlockSpec((dout, din), lambda i: (0, 0)),
        ],
        out_specs=pl.BlockSpec((block_rows, dout), lambda i: (i, 0)),
        out_shape=jax.ShapeDtypeStruct((n, dout), jnp.float32),
    )(agg2, h, wrel, b.reshape(1, -1), wroot)


def _agg_jnp(h, src, dst):
    out = jnp.zeros((2, h.shape[0], h.shape[1]), h.dtype)
    return out.at[0, dst].add(h[src])


def kernel(x, edge_index, Wd_rel, bd_rel, Wd_root, W0_rel, b0_rel, W0_root,
           W1_rel, b1_rel, W1_root, W2_rel, b2_rel, W2_root, p,
           W3_rel, b3_rel, W3_root, Wout, bout):
    src, dst = edge_index[0], edge_index[1]
    x_pad = jnp.zeros((N_PAD, D_FEAT), jnp.float32).at[:N].set(x)

    h0 = _dense0(x_pad, Wd_root, bd_rel)
    agg0 = _agg_jnp(h0, src, dst)
    h1 = _dense(agg0, h0, W0_rel, b0_rel, W0_root)
    agg1 = _agg_jnp(h1, src, dst)
    h2 = _dense(agg1, h1, W1_rel, b1_rel, W1_root)
    agg2 = _agg_jnp(h2, src, dst)
    h3 = _dense(agg2, h2, W2_rel, b2_rel, W2_root)

    score = jnp.tanh((h3[:N] @ p) / jnp.linalg.norm(p))
    idx = jnp.arange(N)
    gt = (score[:, None] > score[None, :]) | (
        (score[:, None] == score[None, :]) & (idx[:, None] < idx[None, :]))
    rank = jnp.sum(gt.astype(jnp.int32), axis=0)
    kept = rank < K
    parity = rank % 2

    hp = jnp.where(kept[:, None], h3[:N] * score[:, None], 0.0)
    hp = jnp.zeros((N_PAD, D_FEAT), jnp.float32).at[:N].set(hp)
    agg3 = _agg_jnp(hp, src, dst)
    h4 = _dense(agg3, hp, W3_rel, b3_rel, W3_root)

    w_even = (kept & (parity == 0)).astype(jnp.float32)
    w_odd = (kept & (parity == 1)).astype(jnp.float32)
    wsel = jnp.zeros((2, N_PAD), jnp.float32)
    wsel = wsel.at[0, :N].set(w_even).at[1, :N].set(w_odd)
    pooled = (wsel @ h4) / (K // 2)
    return Wout @ pooled.reshape(-1) + bout


# speedup vs baseline: 19.1828x; 6.6654x over previous
import functools

import jax
import jax.numpy as jnp
from jax import lax
from jax.experimental import pallas as pl
from jax.experimental.pallas import tpu as pltpu
from jax.experimental.pallas import tpu_sc as plsc

N = 10000
N_PAD = 10240
E = 320000
D_FEAT = 128
CH = 32
K = 6000


def _gelu(x):
    return 0.5 * x * (1.0 + jax.lax.erf(x * 0.7071067811865476))


def _dense0_body(x_ref, w_ref, b_ref, o_ref):
    y = jax.lax.dot_general(x_ref[...], w_ref[...], (((1,), (1,)), ((), ())),
                            preferred_element_type=jnp.float32)
    o_ref[...] = _gelu(y + b_ref[...])


def _dense0(x, w, b, block_rows=2048):
    n = x.shape[0]
    dout = w.shape[0]
    return pl.pallas_call(
        _dense0_body,
        grid=(n // block_rows,),
        in_specs=[
            pl.BlockSpec((block_rows, x.shape[1]), lambda i: (i, 0)),
            pl.BlockSpec((w.shape[0], w.shape[1]), lambda i: (0, 0)),
            pl.BlockSpec((1, dout), lambda i: (0, 0)),
        ],
        out_specs=pl.BlockSpec((block_rows, dout), lambda i: (i, 0)),
        out_shape=jax.ShapeDtypeStruct((n, dout), jnp.float32),
    )(x, w, b.reshape(1, -1))


def _dense_body(agg_ref, h_ref, wrel_ref, b_ref, wroot_ref, o_ref):
    a = agg_ref[0] + agg_ref[1]
    y = jax.lax.dot_general(a, wrel_ref[...], (((1,), (1,)), ((), ())),
                            preferred_element_type=jnp.float32)
    y = y + jax.lax.dot_general(h_ref[...], wroot_ref[...], (((1,), (1,)), ((), ())),
                                preferred_element_type=jnp.float32)
    o_ref[...] = _gelu(y + b_ref[...])


def _dense(agg2, h, wrel, b, wroot, block_rows=2048):
    n = h.shape[0]
    din = h.shape[1]
    dout = wrel.shape[0]
    return pl.pallas_call(
        _dense_body,
        grid=(n // block_rows,),
        in_specs=[
            pl.BlockSpec((2, block_rows, din), lambda i: (0, i, 0)),
            pl.BlockSpec((block_rows, din), lambda i: (i, 0)),
            pl.BlockSpec((dout, din), lambda i: (0, 0)),
            pl.BlockSpec((1, dout), lambda i: (0, 0)),
            pl.BlockSpec((dout, din), lambda i: (0, 0)),
        ],
        out_specs=pl.BlockSpec((block_rows, dout), lambda i: (i, 0)),
        out_shape=jax.ShapeDtypeStruct((n, dout), jnp.float32),
    )(agg2, h, wrel, b.reshape(1, -1), wroot)


_EC = 80
_NW = 32
_NCHUNK = E // _NW // _EC
_RPT = N_PAD // 16


@functools.cache
def _make_agg(d):
    mesh = plsc.VectorSubcoreMesh(core_axis_name="c", subcore_axis_name="s")

    @functools.partial(
        pl.kernel, mesh=mesh,
        compiler_params=pltpu.CompilerParams(use_tc_tiling_on_sc=False),
        out_type=jax.ShapeDtypeStruct((2 * N_PAD, d), jnp.float32),
        scratch_types=[
            pltpu.VMEM((1, _NCHUNK, _EC), jnp.int32),
            pltpu.VMEM((1, _NCHUNK, _EC), jnp.int32),
            pltpu.VMEM((_EC, d), jnp.float32),
            pltpu.VMEM_SHARED((N_PAD, d), jnp.float32),
            pltpu.SemaphoreType.DMA,
        ],
    )
    def agg(h_hbm, s2d_hbm, d2d_hbm, z_hbm, out_hbm, sidx, didx, rows, acc, sem):
        c = lax.axis_index("c")
        s = lax.axis_index("s")
        wid = s * 2 + c
        pltpu.sync_copy(z_hbm.at[pl.ds(s * _RPT, _RPT)],
                        acc.at[pl.ds(s * _RPT, _RPT)])
        pltpu.sync_copy(s2d_hbm.at[pl.ds(wid, 1)], sidx)
        pltpu.sync_copy(d2d_hbm.at[pl.ds(wid, 1)], didx)
        plsc.subcore_barrier()

        def body(j, carry):
            pltpu.async_copy(h_hbm.at[sidx.at[0, j]], rows, sem).wait()
            pltpu.sync_copy(rows, acc.at[didx.at[0, j]], add=True)
            return carry

        lax.fori_loop(0, _NCHUNK, body, 0)
        plsc.subcore_barrier()
        pltpu.sync_copy(acc.at[pl.ds(s * _RPT, _RPT)],
                        out_hbm.at[pl.ds(c * N_PAD + s * _RPT, _RPT)])

    return agg


def _agg_sc(h, src2d, dst2d, zeros):
    d = h.shape[1]
    out = _make_agg(d)(h, src2d, dst2d, zeros)
    return out.reshape(2, N_PAD, d)


def kernel(x, edge_index, Wd_rel, bd_rel, Wd_root, W0_rel, b0_rel, W0_root,
           W1_rel, b1_rel, W1_root, W2_rel, b2_rel, W2_root, p,
           W3_rel, b3_rel, W3_root, Wout, bout):
    src2d = edge_index[0].reshape(_NW, _NCHUNK, _EC)
    dst2d = edge_index[1].reshape(_NW, _NCHUNK, _EC)
    x_pad = jnp.zeros((N_PAD, D_FEAT), jnp.float32).at[:N].set(x)

    Wd_root8 = jnp.zeros((8, D_FEAT), jnp.float32).at[:4].set(Wd_root)
    bd8 = jnp.zeros((8,), jnp.float32).at[:4].set(bd_rel)
    W0_rel8 = jnp.zeros((CH, 8), jnp.float32).at[:, :4].set(W0_rel)
    W0_root8 = jnp.zeros((CH, 8), jnp.float32).at[:, :4].set(W0_root)

    h0 = _dense0(x_pad, Wd_root8, bd8)
    agg0 = _agg_sc(h0, src2d, dst2d, jnp.zeros((N_PAD, 8), jnp.float32))
    h1 = _dense(agg0, h0, W0_rel8, b0_rel, W0_root8)
    agg1 = _agg_sc(h1, src2d, dst2d, jnp.zeros((N_PAD, 32), jnp.float32))
    h2 = _dense(agg1, h1, W1_rel, b1_rel, W1_root)
    agg2 = _agg_sc(h2, src2d, dst2d, jnp.zeros((N_PAD, 64), jnp.float32))
    h3 = _dense(agg2, h2, W2_rel, b2_rel, W2_root)

    score = jnp.tanh((h3[:N] @ p) / jnp.linalg.norm(p))
    idx = jnp.arange(N)
    gt = (score[:, None] > score[None, :]) | (
        (score[:, None] == score[None, :]) & (idx[:, None] < idx[None, :]))
    rank = jnp.sum(gt.astype(jnp.int32), axis=0)
    kept = rank < K
    parity = rank % 2

    hp = jnp.where(kept[:, None], h3[:N] * score[:, None], 0.0)
    hp = jnp.zeros((N_PAD, D_FEAT), jnp.float32).at[:N].set(hp)
    agg3 = _agg_sc(hp, src2d, dst2d, jnp.zeros((N_PAD, D_FEAT), jnp.float32))
    h4 = _dense(agg3, hp, W3_rel, b3_rel, W3_root)

    w_even = (kept & (parity == 0)).astype(jnp.float32)
    w_odd = (kept & (parity == 1)).astype(jnp.float32)
    wsel = jnp.zeros((2, N_PAD), jnp.float32)
    wsel = wsel.at[0, :N].set(w_even).at[1, :N].set(w_odd)
    pooled = (wsel @ h4) / (K // 2)
    return Wout @ pooled.reshape(-1) + bout


# speedup vs baseline: 20.1686x; 1.0514x over previous
import functools

import jax
import jax.numpy as jnp
from jax import lax
from jax.experimental import pallas as pl
from jax.experimental.pallas import tpu as pltpu
from jax.experimental.pallas import tpu_sc as plsc

N = 10000
N_PAD = 10240
E = 320000
D_FEAT = 128
CH = 32
K = 6000


def _gelu(x):
    return 0.5 * x * (1.0 + jax.lax.erf(x * 0.7071067811865476))


def _dense0_body(x_ref, w_ref, b_ref, o_ref):
    y = jax.lax.dot_general(x_ref[...], w_ref[...], (((1,), (1,)), ((), ())),
                            preferred_element_type=jnp.float32)
    o_ref[...] = _gelu(y + b_ref[...])


def _dense0(x, w, b, block_rows=2048):
    n = x.shape[0]
    dout = w.shape[0]
    return pl.pallas_call(
        _dense0_body,
        grid=(n // block_rows,),
        in_specs=[
            pl.BlockSpec((block_rows, x.shape[1]), lambda i: (i, 0)),
            pl.BlockSpec((w.shape[0], w.shape[1]), lambda i: (0, 0)),
            pl.BlockSpec((1, dout), lambda i: (0, 0)),
        ],
        out_specs=pl.BlockSpec((block_rows, dout), lambda i: (i, 0)),
        out_shape=jax.ShapeDtypeStruct((n, dout), jnp.float32),
    )(x, w, b.reshape(1, -1))


def _dense_body(agg_ref, h_ref, wrel_ref, b_ref, wroot_ref, o_ref):
    a = agg_ref[0] + agg_ref[1]
    y = jax.lax.dot_general(a, wrel_ref[...], (((1,), (1,)), ((), ())),
                            preferred_element_type=jnp.float32)
    y = y + jax.lax.dot_general(h_ref[...], wroot_ref[...], (((1,), (1,)), ((), ())),
                                preferred_element_type=jnp.float32)
    o_ref[...] = _gelu(y + b_ref[...])


def _dense(agg2, h, wrel, b, wroot, block_rows=2048):
    n = h.shape[0]
    din = h.shape[1]
    dout = wrel.shape[0]
    return pl.pallas_call(
        _dense_body,
        grid=(n // block_rows,),
        in_specs=[
            pl.BlockSpec((2, block_rows, din), lambda i: (0, i, 0)),
            pl.BlockSpec((block_rows, din), lambda i: (i, 0)),
            pl.BlockSpec((dout, din), lambda i: (0, 0)),
            pl.BlockSpec((1, dout), lambda i: (0, 0)),
            pl.BlockSpec((dout, din), lambda i: (0, 0)),
        ],
        out_specs=pl.BlockSpec((block_rows, dout), lambda i: (i, 0)),
        out_shape=jax.ShapeDtypeStruct((n, dout), jnp.float32),
    )(agg2, h, wrel, b.reshape(1, -1), wroot)


def _rank_body(s_ref, col_ref, m_ref, we_ref, wo_ref):
    col = col_ref[...].reshape(128, 1)
    c = pl.program_id(0)
    A = jnp.broadcast_to(col, (128, 128))
    li = lax.broadcasted_iota(jnp.int32, (128, 128), 0)
    lj = lax.broadcasted_iota(jnp.int32, (128, 128), 1)
    lmask = lj < li

    def ge_body(q, acc):
        B = jnp.broadcast_to(s_ref[pl.ds(q, 1), :], (128, 128))
        return acc + jnp.where(B >= A, 1.0, 0.0)

    def gt_body(q, acc):
        B = jnp.broadcast_to(s_ref[pl.ds(q, 1), :], (128, 128))
        return acc + jnp.where(B > A, 1.0, 0.0)

    acc = lax.fori_loop(0, c, ge_body, jnp.zeros((128, 128), jnp.float32))
    acc = lax.fori_loop(c + 1, 80, gt_body, acc)
    Bd = jnp.broadcast_to(s_ref[pl.ds(c, 1), :], (128, 128))
    acc = acc + jnp.where((Bd > A) | ((Bd == A) & lmask), 1.0, 0.0)
    rank = jnp.sum(acc, axis=1, keepdims=True).astype(jnp.int32)
    kept = rank < K
    m_ref[...] = jnp.where(kept, col, 0.0).reshape(1, 128, 1)
    we_ref[...] = jnp.where(kept & (rank % 2 == 0), 1.0, 0.0).reshape(1, 128, 1)
    wo_ref[...] = jnp.where(kept & (rank % 2 == 1), 1.0, 0.0).reshape(1, 128, 1)


def _rank(s2d):
    s3d = s2d.reshape(80, 128, 1)
    outs = pl.pallas_call(
        _rank_body,
        grid=(80,),
        in_specs=[
            pl.BlockSpec((80, 128), lambda c: (0, 0)),
            pl.BlockSpec((1, 128, 1), lambda c: (c, 0, 0)),
        ],
        out_specs=[pl.BlockSpec((1, 128, 1), lambda c: (c, 0, 0))] * 3,
        out_shape=[jax.ShapeDtypeStruct((80, 128, 1), jnp.float32)] * 3,
    )(s2d, s3d)
    return outs


def _dense3_body(agg_ref, h_ref, wrel_ref, b_ref, wroot_ref, p_ref, o_ref, s_ref):
    a = agg_ref[0] + agg_ref[1]
    y = jax.lax.dot_general(a, wrel_ref[...], (((1,), (1,)), ((), ())),
                            preferred_element_type=jnp.float32)
    y = y + jax.lax.dot_general(h_ref[...], wroot_ref[...], (((1,), (1,)), ((), ())),
                                preferred_element_type=jnp.float32)
    h3 = _gelu(y + b_ref[...])
    o_ref[...] = h3
    pn = p_ref[...] * jax.lax.rsqrt(jnp.sum(p_ref[...] * p_ref[...]))
    s_ref[...] = jnp.tanh(jax.lax.dot_general(
        h3, pn, (((1,), (1,)), ((), ())), preferred_element_type=jnp.float32))


def _dense3(agg2, h, wrel, b, wroot, p, block_rows=2048):
    n = h.shape[0]
    din = h.shape[1]
    dout = wrel.shape[0]
    return pl.pallas_call(
        _dense3_body,
        grid=(n // block_rows,),
        in_specs=[
            pl.BlockSpec((2, block_rows, din), lambda i: (0, i, 0)),
            pl.BlockSpec((block_rows, din), lambda i: (i, 0)),
            pl.BlockSpec((dout, din), lambda i: (0, 0)),
            pl.BlockSpec((1, dout), lambda i: (0, 0)),
            pl.BlockSpec((dout, din), lambda i: (0, 0)),
            pl.BlockSpec((1, dout), lambda i: (0, 0)),
        ],
        out_specs=[pl.BlockSpec((block_rows, dout), lambda i: (i, 0)),
                   pl.BlockSpec((block_rows, 1), lambda i: (i, 0))],
        out_shape=[jax.ShapeDtypeStruct((n, dout), jnp.float32),
                   jax.ShapeDtypeStruct((n, 1), jnp.float32)],
    )(agg2, h, wrel, b.reshape(1, -1), wroot, p.reshape(1, -1))


def _dense4_body(agg_ref, h_ref, wrel_ref, b_ref, wroot_ref, wsel_ref,
                 wout_ref, bout_ref, pool_ref, o_ref):
    i = pl.program_id(0)
    a = agg_ref[0] + agg_ref[1]
    y = jax.lax.dot_general(a, wrel_ref[...], (((1,), (1,)), ((), ())),
                            preferred_element_type=jnp.float32)
    y = y + jax.lax.dot_general(h_ref[...], wroot_ref[...], (((1,), (1,)), ((), ())),
                                preferred_element_type=jnp.float32)
    h4 = _gelu(y + b_ref[...])
    part = jax.lax.dot_general(wsel_ref[...], h4, (((1,), (0,)), ((), ())),
                               preferred_element_type=jnp.float32)

    @pl.when(i == 0)
    def _():
        pool_ref[...] = jnp.zeros_like(pool_ref)

    pool_ref[...] += part

    @pl.when(i == pl.num_programs(0) - 1)
    def _():
        pooled = pool_ref[...] / (K // 2)
        o_ref[...] = (jnp.sum(pooled * wout_ref[...], keepdims=True)[:, :1]
                      + bout_ref[...])


def _dense4(agg2, h, wrel, b, wroot, wsel, wout2, bout, block_rows=2048):
    n = h.shape[0]
    din = h.shape[1]
    dout = wrel.shape[0]
    pooled, out = pl.pallas_call(
        _dense4_body,
        grid=(n // block_rows,),
        in_specs=[
            pl.BlockSpec((2, block_rows, din), lambda i: (0, i, 0)),
            pl.BlockSpec((block_rows, din), lambda i: (i, 0)),
            pl.BlockSpec((dout, din), lambda i: (0, 0)),
            pl.BlockSpec((1, dout), lambda i: (0, 0)),
            pl.BlockSpec((dout, din), lambda i: (0, 0)),
            pl.BlockSpec((2, block_rows), lambda i: (0, i)),
            pl.BlockSpec((2, dout), lambda i: (0, 0)),
            pl.BlockSpec((1, 1), lambda i: (0, 0)),
        ],
        out_specs=[pl.BlockSpec((2, dout), lambda i: (0, 0)),
                   pl.BlockSpec((1, 1), lambda i: (0, 0))],
        out_shape=[jax.ShapeDtypeStruct((2, dout), jnp.float32),
                   jax.ShapeDtypeStruct((1, 1), jnp.float32)],
    )(agg2, h, wrel, b.reshape(1, -1), wroot, wsel, wout2,
      bout.reshape(1, 1))
    return out


_EC = 80
_NW = 32
_NCHUNK = E // _NW // _EC
_RPT = N_PAD // 16


@functools.cache
def _make_agg(d):
    mesh = plsc.VectorSubcoreMesh(core_axis_name="c", subcore_axis_name="s")

    @functools.partial(
        pl.kernel, mesh=mesh,
        compiler_params=pltpu.CompilerParams(use_tc_tiling_on_sc=False),
        out_type=jax.ShapeDtypeStruct((2 * N_PAD, d), jnp.float32),
        scratch_types=[
            pltpu.VMEM((1, _NCHUNK, _EC), jnp.int32),
            pltpu.VMEM((1, _NCHUNK, _EC), jnp.int32),
            pltpu.VMEM((_EC, d), jnp.float32),
            pltpu.VMEM_SHARED((N_PAD, d), jnp.float32),
            pltpu.SemaphoreType.DMA,
        ],
    )
    def agg(h_hbm, s2d_hbm, d2d_hbm, z_hbm, out_hbm, sidx, didx, rows, acc, sem):
        c = lax.axis_index("c")
        s = lax.axis_index("s")
        wid = s * 2 + c
        pltpu.sync_copy(z_hbm.at[pl.ds(s * _RPT, _RPT)],
                        acc.at[pl.ds(s * _RPT, _RPT)])
        pltpu.sync_copy(s2d_hbm.at[pl.ds(wid, 1)], sidx)
        pltpu.sync_copy(d2d_hbm.at[pl.ds(wid, 1)], didx)
        plsc.subcore_barrier()

        def body(j, carry):
            pltpu.async_copy(h_hbm.at[sidx.at[0, j]], rows, sem).wait()
            pltpu.sync_copy(rows, acc.at[didx.at[0, j]], add=True)
            return carry

        lax.fori_loop(0, _NCHUNK, body, 0)
        plsc.subcore_barrier()
        pltpu.sync_copy(acc.at[pl.ds(s * _RPT, _RPT)],
                        out_hbm.at[pl.ds(c * N_PAD + s * _RPT, _RPT)])

    return agg


def _agg_sc(h, src2d, dst2d, zeros):
    d = h.shape[1]
    out = _make_agg(d)(h, src2d, dst2d, zeros)
    return out.reshape(2, N_PAD, d)


def kernel(x, edge_index, Wd_rel, bd_rel, Wd_root, W0_rel, b0_rel, W0_root,
           W1_rel, b1_rel, W1_root, W2_rel, b2_rel, W2_root, p,
           W3_rel, b3_rel, W3_root, Wout, bout):
    src2d = edge_index[0].reshape(_NW, _NCHUNK, _EC)
    dst2d = edge_index[1].reshape(_NW, _NCHUNK, _EC)
    x_pad = jnp.zeros((N_PAD, D_FEAT), jnp.float32).at[:N].set(x)

    Wd_root8 = jnp.zeros((8, D_FEAT), jnp.float32).at[:4].set(Wd_root)
    bd8 = jnp.zeros((8,), jnp.float32).at[:4].set(bd_rel)
    W0_rel8 = jnp.zeros((CH, 8), jnp.float32).at[:, :4].set(W0_rel)
    W0_root8 = jnp.zeros((CH, 8), jnp.float32).at[:, :4].set(W0_root)

    h0 = _dense0(x_pad, Wd_root8, bd8)
    agg0 = _agg_sc(h0, src2d, dst2d, jnp.zeros((N_PAD, 8), jnp.float32))
    h1 = _dense(agg0, h0, W0_rel8, b0_rel, W0_root8)
    agg1 = _agg_sc(h1, src2d, dst2d, jnp.zeros((N_PAD, 32), jnp.float32))
    h2 = _dense(agg1, h1, W1_rel, b1_rel, W1_root)
    agg2 = _agg_sc(h2, src2d, dst2d, jnp.zeros((N_PAD, 64), jnp.float32))
    h3, score = _dense3(agg2, h2, W2_rel, b2_rel, W2_root, p)

    score = score.reshape(N_PAD).at[N:].set(-jnp.inf)
    s2d = score.reshape(80, 128)
    m3, we3, wo3 = _rank(s2d)
    m = m3.reshape(N_PAD, 1)
    wsel = jnp.stack([we3.reshape(N_PAD), wo3.reshape(N_PAD)])

    hp = h3 * m
    agg3 = _agg_sc(hp, src2d, dst2d, jnp.zeros((N_PAD, D_FEAT), jnp.float32))
    out = _dense4(agg3, hp, W3_rel, b3_rel, W3_root, wsel,
                  Wout.reshape(2, 8 * CH), bout)
    return out.reshape(1)


# speedup vs baseline: 29.8068x; 1.4779x over previous
import functools

import jax
import jax.numpy as jnp
from jax import lax
from jax.experimental import pallas as pl
from jax.experimental.pallas import tpu as pltpu
from jax.experimental.pallas import tpu_sc as plsc

N = 10000
N_PAD = 10240
E = 320000
D_FEAT = 128
CH = 32
K = 6000


def _gelu(x):
    return 0.5 * x * (1.0 + jax.lax.erf(x * 0.7071067811865476))


def _dense0_body(x_ref, w_ref, b_ref, o_ref):
    y = jax.lax.dot_general(x_ref[...], w_ref[...], (((1,), (1,)), ((), ())),
                            preferred_element_type=jnp.float32)
    o_ref[...] = _gelu(y + b_ref[...])


def _dense0(x, w, b, block_rows=2048):
    n = x.shape[0]
    dout = w.shape[0]
    return pl.pallas_call(
        _dense0_body,
        grid=(n // block_rows,),
        in_specs=[
            pl.BlockSpec((block_rows, x.shape[1]), lambda i: (i, 0)),
            pl.BlockSpec((w.shape[0], w.shape[1]), lambda i: (0, 0)),
            pl.BlockSpec((1, dout), lambda i: (0, 0)),
        ],
        out_specs=pl.BlockSpec((block_rows, dout), lambda i: (i, 0)),
        out_shape=jax.ShapeDtypeStruct((n, dout), jnp.float32),
    )(x, w, b.reshape(1, -1))


def _dense_body(agg_ref, h_ref, wrel_ref, b_ref, wroot_ref, o_ref):
    a = agg_ref[0] + agg_ref[1]
    y = jax.lax.dot_general(a, wrel_ref[...], (((1,), (1,)), ((), ())),
                            preferred_element_type=jnp.float32)
    y = y + jax.lax.dot_general(h_ref[...], wroot_ref[...], (((1,), (1,)), ((), ())),
                                preferred_element_type=jnp.float32)
    o_ref[...] = _gelu(y + b_ref[...])


def _dense(agg2, h, wrel, b, wroot, block_rows=2048):
    n = h.shape[0]
    din = h.shape[1]
    dout = wrel.shape[0]
    return pl.pallas_call(
        _dense_body,
        grid=(n // block_rows,),
        in_specs=[
            pl.BlockSpec((2, block_rows, din), lambda i: (0, i, 0)),
            pl.BlockSpec((block_rows, din), lambda i: (i, 0)),
            pl.BlockSpec((dout, din), lambda i: (0, 0)),
            pl.BlockSpec((1, dout), lambda i: (0, 0)),
            pl.BlockSpec((dout, din), lambda i: (0, 0)),
        ],
        out_specs=pl.BlockSpec((block_rows, dout), lambda i: (i, 0)),
        out_shape=jax.ShapeDtypeStruct((n, dout), jnp.float32),
    )(agg2, h, wrel, b.reshape(1, -1), wroot)


def _rank_body(s_ref, col_ref, m_ref, we_ref, wo_ref):
    col = col_ref[...].reshape(128, 1)
    c = pl.program_id(0)
    A = jnp.broadcast_to(col, (128, 128))
    li = lax.broadcasted_iota(jnp.int32, (128, 128), 0)
    lj = lax.broadcasted_iota(jnp.int32, (128, 128), 1)
    lmask = lj < li

    def ge_body(q, acc):
        B = jnp.broadcast_to(s_ref[pl.ds(q, 1), :], (128, 128))
        return acc + jnp.where(B >= A, 1.0, 0.0)

    def gt_body(q, acc):
        B = jnp.broadcast_to(s_ref[pl.ds(q, 1), :], (128, 128))
        return acc + jnp.where(B > A, 1.0, 0.0)

    acc = lax.fori_loop(0, c, ge_body, jnp.zeros((128, 128), jnp.float32))
    acc = lax.fori_loop(c + 1, 80, gt_body, acc)
    Bd = jnp.broadcast_to(s_ref[pl.ds(c, 1), :], (128, 128))
    acc = acc + jnp.where((Bd > A) | ((Bd == A) & lmask), 1.0, 0.0)
    rank = jnp.sum(acc, axis=1, keepdims=True).astype(jnp.int32)
    kept = rank < K
    m_ref[...] = jnp.where(kept, col, 0.0).reshape(1, 128, 1)
    we_ref[...] = jnp.where(kept & (rank % 2 == 0), 1.0, 0.0).reshape(1, 128, 1)
    wo_ref[...] = jnp.where(kept & (rank % 2 == 1), 1.0, 0.0).reshape(1, 128, 1)


def _rank(s2d):
    s3d = s2d.reshape(80, 128, 1)
    outs = pl.pallas_call(
        _rank_body,
        grid=(80,),
        in_specs=[
            pl.BlockSpec((80, 128), lambda c: (0, 0)),
            pl.BlockSpec((1, 128, 1), lambda c: (c, 0, 0)),
        ],
        out_specs=[pl.BlockSpec((1, 128, 1), lambda c: (c, 0, 0))] * 3,
        out_shape=[jax.ShapeDtypeStruct((80, 128, 1), jnp.float32)] * 3,
    )(s2d, s3d)
    return outs


def _dense3_body(agg_ref, h_ref, wrel_ref, b_ref, wroot_ref, p_ref, o_ref, s_ref):
    a = agg_ref[0] + agg_ref[1]
    y = jax.lax.dot_general(a, wrel_ref[...], (((1,), (1,)), ((), ())),
                            preferred_element_type=jnp.float32)
    y = y + jax.lax.dot_general(h_ref[...], wroot_ref[...], (((1,), (1,)), ((), ())),
                                preferred_element_type=jnp.float32)
    h3 = _gelu(y + b_ref[...])
    o_ref[...] = h3
    pn = p_ref[...] * jax.lax.rsqrt(jnp.sum(p_ref[...] * p_ref[...]))
    s_ref[...] = jnp.tanh(jax.lax.dot_general(
        h3, pn, (((1,), (1,)), ((), ())), preferred_element_type=jnp.float32))


def _dense3(agg2, h, wrel, b, wroot, p, block_rows=2048):
    n = h.shape[0]
    din = h.shape[1]
    dout = wrel.shape[0]
    return pl.pallas_call(
        _dense3_body,
        grid=(n // block_rows,),
        in_specs=[
            pl.BlockSpec((2, block_rows, din), lambda i: (0, i, 0)),
            pl.BlockSpec((block_rows, din), lambda i: (i, 0)),
            pl.BlockSpec((dout, din), lambda i: (0, 0)),
            pl.BlockSpec((1, dout), lambda i: (0, 0)),
            pl.BlockSpec((dout, din), lambda i: (0, 0)),
            pl.BlockSpec((1, dout), lambda i: (0, 0)),
        ],
        out_specs=[pl.BlockSpec((block_rows, dout), lambda i: (i, 0)),
                   pl.BlockSpec((block_rows, 1), lambda i: (i, 0))],
        out_shape=[jax.ShapeDtypeStruct((n, dout), jnp.float32),
                   jax.ShapeDtypeStruct((n, 1), jnp.float32)],
    )(agg2, h, wrel, b.reshape(1, -1), wroot, p.reshape(1, -1))


def _dense4_body(agg_ref, h_ref, wrel_ref, b_ref, wroot_ref, wsel_ref,
                 wout_ref, bout_ref, pool_ref, o_ref):
    i = pl.program_id(0)
    a = agg_ref[0] + agg_ref[1]
    y = jax.lax.dot_general(a, wrel_ref[...], (((1,), (1,)), ((), ())),
                            preferred_element_type=jnp.float32)
    y = y + jax.lax.dot_general(h_ref[...], wroot_ref[...], (((1,), (1,)), ((), ())),
                                preferred_element_type=jnp.float32)
    h4 = _gelu(y + b_ref[...])
    part = jax.lax.dot_general(wsel_ref[...], h4, (((1,), (0,)), ((), ())),
                               preferred_element_type=jnp.float32)

    @pl.when(i == 0)
    def _():
        pool_ref[...] = jnp.zeros_like(pool_ref)

    pool_ref[...] += part

    @pl.when(i == pl.num_programs(0) - 1)
    def _():
        pooled = pool_ref[...] / (K // 2)
        o_ref[...] = (jnp.sum(pooled * wout_ref[...], keepdims=True)[:, :1]
                      + bout_ref[...])


def _dense4(agg2, h, wrel, b, wroot, wsel, wout2, bout, block_rows=2048):
    n = h.shape[0]
    din = h.shape[1]
    dout = wrel.shape[0]
    pooled, out = pl.pallas_call(
        _dense4_body,
        grid=(n // block_rows,),
        in_specs=[
            pl.BlockSpec((2, block_rows, din), lambda i: (0, i, 0)),
            pl.BlockSpec((block_rows, din), lambda i: (i, 0)),
            pl.BlockSpec((dout, din), lambda i: (0, 0)),
            pl.BlockSpec((1, dout), lambda i: (0, 0)),
            pl.BlockSpec((dout, din), lambda i: (0, 0)),
            pl.BlockSpec((2, block_rows), lambda i: (0, i)),
            pl.BlockSpec((2, dout), lambda i: (0, 0)),
            pl.BlockSpec((1, 1), lambda i: (0, 0)),
        ],
        out_specs=[pl.BlockSpec((2, dout), lambda i: (0, 0)),
                   pl.BlockSpec((1, 1), lambda i: (0, 0))],
        out_shape=[jax.ShapeDtypeStruct((2, dout), jnp.float32),
                   jax.ShapeDtypeStruct((1, 1), jnp.float32)],
    )(agg2, h, wrel, b.reshape(1, -1), wroot, wsel, wout2,
      bout.reshape(1, 1))
    return out


_EC = 80
_NW = 32
_NCHUNK = E // _NW // _EC
_RPT = N_PAD // 16


@functools.cache
def _make_agg(d):
    mesh = plsc.VectorSubcoreMesh(core_axis_name="c", subcore_axis_name="s")

    @functools.partial(
        pl.kernel, mesh=mesh,
        compiler_params=pltpu.CompilerParams(use_tc_tiling_on_sc=False),
        out_type=jax.ShapeDtypeStruct((2 * N_PAD, d), jnp.float32),
        scratch_types=[
            pltpu.VMEM((1, _NCHUNK, _EC), jnp.int32),
            pltpu.VMEM((1, _NCHUNK, _EC), jnp.int32),
            pltpu.VMEM((2, _EC, d), jnp.float32),
            pltpu.VMEM_SHARED((N_PAD, d), jnp.float32),
            pltpu.SemaphoreType.DMA,
            pltpu.SemaphoreType.DMA,
            pltpu.SemaphoreType.DMA,
        ],
    )
    def agg(h_hbm, s2d_hbm, d2d_hbm, z_hbm, out_hbm, sidx, didx, rows, acc,
            g0, g1, semz):
        c = lax.axis_index("c")
        s = lax.axis_index("s")
        wid = s * 2 + c
        zc = pltpu.async_copy(z_hbm.at[pl.ds(s * _RPT, _RPT)],
                              acc.at[pl.ds(s * _RPT, _RPT)], semz)
        c1 = pltpu.async_copy(s2d_hbm.at[pl.ds(wid, 1)], sidx, g0)
        c2 = pltpu.async_copy(d2d_hbm.at[pl.ds(wid, 1)], didx, g1)
        c1.wait()
        c2.wait()
        zc.wait()
        plsc.subcore_barrier()

        r0 = rows.at[0]
        r1 = rows.at[1]
        pltpu.async_copy(h_hbm.at[sidx.at[0, 0]], r0, g0)

        def body(g, carry):
            j0 = 2 * g
            j1 = j0 + 1

            @pl.when(j1 < _NCHUNK)
            def _():
                pltpu.async_copy(h_hbm.at[sidx.at[0, j1]], r1, g1)

            pltpu.make_async_copy(h_hbm.at[sidx.at[0, j0]], r0, g0).wait()
            pltpu.sync_copy(r0, acc.at[didx.at[0, j0]], add=True)

            @pl.when(j0 + 2 < _NCHUNK)
            def _():
                pltpu.async_copy(h_hbm.at[sidx.at[0, j0 + 2]], r0, g0)

            @pl.when(j1 < _NCHUNK)
            def _():
                pltpu.make_async_copy(h_hbm.at[sidx.at[0, j1]], r1, g1).wait()
                pltpu.sync_copy(r1, acc.at[didx.at[0, j1]], add=True)

            return carry

        lax.fori_loop(0, (_NCHUNK + 1) // 2, body, 0)
        plsc.subcore_barrier()
        pltpu.sync_copy(acc.at[pl.ds(s * _RPT, _RPT)],
                        out_hbm.at[pl.ds(c * N_PAD + s * _RPT, _RPT)])

    return agg


def _agg_sc(h, src2d, dst2d, zeros):
    d = h.shape[1]
    out = _make_agg(d)(h, src2d, dst2d, zeros)
    return out.reshape(2, N_PAD, d)


def kernel(x, edge_index, Wd_rel, bd_rel, Wd_root, W0_rel, b0_rel, W0_root,
           W1_rel, b1_rel, W1_root, W2_rel, b2_rel, W2_root, p,
           W3_rel, b3_rel, W3_root, Wout, bout):
    src2d = edge_index[0].reshape(_NW, _NCHUNK, _EC)
    dst2d = edge_index[1].reshape(_NW, _NCHUNK, _EC)
    x_pad = jnp.zeros((N_PAD, D_FEAT), jnp.float32).at[:N].set(x)

    Wd_root8 = jnp.zeros((8, D_FEAT), jnp.float32).at[:4].set(Wd_root)
    bd8 = jnp.zeros((8,), jnp.float32).at[:4].set(bd_rel)
    W0_rel8 = jnp.zeros((CH, 8), jnp.float32).at[:, :4].set(W0_rel)
    W0_root8 = jnp.zeros((CH, 8), jnp.float32).at[:, :4].set(W0_root)

    h0 = _dense0(x_pad, Wd_root8, bd8)
    agg0 = _agg_sc(h0, src2d, dst2d, jnp.zeros((N_PAD, 8), jnp.float32))
    h1 = _dense(agg0, h0, W0_rel8, b0_rel, W0_root8)
    agg1 = _agg_sc(h1, src2d, dst2d, jnp.zeros((N_PAD, 32), jnp.float32))
    h2 = _dense(agg1, h1, W1_rel, b1_rel, W1_root)
    agg2 = _agg_sc(h2, src2d, dst2d, jnp.zeros((N_PAD, 64), jnp.float32))
    h3, score = _dense3(agg2, h2, W2_rel, b2_rel, W2_root, p)

    score = score.reshape(N_PAD).at[N:].set(-jnp.inf)
    s2d = score.reshape(80, 128)
    m3, we3, wo3 = _rank(s2d)
    m = m3.reshape(N_PAD, 1)
    wsel = jnp.stack([we3.reshape(N_PAD), wo3.reshape(N_PAD)])

    hp = h3 * m
    agg3 = _agg_sc(hp, src2d, dst2d, jnp.zeros((N_PAD, D_FEAT), jnp.float32))
    out = _dense4(agg3, hp, W3_rel, b3_rel, W3_root, wsel,
                  Wout.reshape(2, 8 * CH), bout)
    return out.reshape(1)
